# baseline (speedup 1.0000x reference)
import functools

import jax
import jax.numpy as jnp
from jax import lax
from jax.experimental import pallas as pl

_THW = 1024
_HW = 256
_FN = 128
_T = 4


def _tc_body(x_ref, a_ref, b_ref, c_ref, bc_ref, out_ref):
    f32 = jnp.float32
    xin = x_ref[0]
    xT = jnp.concatenate([xin[f * _FN:(f + 1) * _FN, :] for f in range(_T)],
                         axis=1)
    xx = xT * xT
    ones = jnp.ones((1, _FN), f32)
    sq_row = lax.dot_general(ones, xx, (((1,), (0,)), ((), ())),
                             preferred_element_type=f32)
    sq_col = lax.dot_general(xx, ones, (((0,), (1,)), ((), ())),
                             preferred_element_type=f32)
    g = lax.dot_general(xT, xT, (((0,), (0,)), ((), ())),
                        preferred_element_type=f32)
    d2 = sq_col + sq_row - 2.0 * g
    d2 = jnp.clip(d2, 1e-5, 100.0)

    rio = lax.broadcasted_iota(jnp.int32, (_THW, _THW), 0)
    cio = lax.broadcasted_iota(jnp.int32, (_THW, _THW), 1)
    same_frame = (rio // _HW) == (cio // _HW)
    d2 = jnp.where(same_frame, 1e9, d2)

    r1 = lax.broadcasted_iota(jnp.int32, (1, _THW), 1)
    c0 = c_ref[:, 0:1]
    c1 = c_ref[:, 1:2]
    c2 = c_ref[:, 2:3]
    in_t = ((r1 // 16) * 4).astype(f32)
    in_h = (r1 % 16).astype(f32)
    p_t = (r1 // _HW).astype(f32) * 0.25
    p_h = ((r1 // 16) % 16).astype(f32)
    p_w = (r1 % 16).astype(f32)
    dn_tm = (((0,), (0,)), ((), ()))
    z_t = (lax.dot_general(a_ref[...], xT, dn_tm, preferred_element_type=f32)
           + bc_ref[...] + c0 * in_t + c1 * in_h)
    y_t = (lax.dot_general(b_ref[...], xT, dn_tm, preferred_element_type=f32)
           + c0 * p_t + c1 * p_h + c2 * p_w)

    acc = None
    cur = d2
    for _ in range(3):
        m = jnp.min(cur, axis=1, keepdims=True)
        cand = jnp.where(cur == m, cio, 2048)
        ik = jnp.min(cand, axis=1, keepdims=True)
        oh = (cio == ik).astype(f32)
        gk = lax.dot_general(y_t, oh, (((1,), (1,)), ((), ())),
                             preferred_element_type=f32)
        acc = gk if acc is None else jnp.maximum(acc, gk)
        cur = jnp.where(cio == ik, 1e9, cur)

    res = z_t + acc
    for f in range(_T):
        out_ref[0, f] = res[:, f * _HW:(f + 1) * _HW]


def kernel(input, W1, b1, W2, b2, W3, b3):
    bs, t, fn, h, w = input.shape
    hw = h * w
    xn = input.reshape(bs, t * fn, hw)

    M = W2.T @ W3.T
    Wc = W1.T @ M
    A = Wc[:fn]
    B = Wc[fn:2 * fn]
    Ct = jnp.zeros((fn, 8), jnp.float32).at[:, :3].set(Wc[2 * fn:].T)
    bc = (b1 @ M + b2 @ W3.T + b3).reshape(fn, 1)

    out = pl.pallas_call(
        _tc_body,
        grid=(bs,),
        in_specs=[
            pl.BlockSpec((1, t * fn, hw), lambda i: (i, 0, 0)),
            pl.BlockSpec((fn, fn), lambda i: (0, 0)),
            pl.BlockSpec((fn, fn), lambda i: (0, 0)),
            pl.BlockSpec((fn, 8), lambda i: (0, 0)),
            pl.BlockSpec((fn, 1), lambda i: (0, 0)),
        ],
        out_specs=pl.BlockSpec((1, t, fn, hw), lambda i: (i, 0, 0, 0)),
        out_shape=jax.ShapeDtypeStruct((bs, t, fn, hw), jnp.float32),
    )(xn, A, B, Ct, bc)

    return out.reshape(bs, t, fn, h, w)
